# baseline (speedup 1.0000x reference)
import functools
import math

import jax
import jax.numpy as jnp
from jax.experimental import pallas as pl

_RADIUS = 0.1
_NB = 8
_BM = 256
_BN = 256


def _tile_body(gat_ref, gbt_ref, fb_ref, w_ref, out_ref):
    j = pl.program_id(2)

    ga = gat_ref[0]
    gb = gbt_ref[0]
    d2 = jnp.zeros((ga.shape[1], gb.shape[1]), jnp.float32)
    for c in range(3):
        dc = ga[c][:, None] - gb[c][None, :]
        d2 = d2 + dc * dc
    mask = d2 < _RADIUS * _RADIUS
    r = jnp.minimum(jnp.sqrt(d2 + 1e-12), 1.25 * _RADIUS)

    inv_sigma = _NB / _RADIUS
    step = _RADIUS / (_NB - 1)
    t0 = r * inv_sigma
    phi = jnp.where(mask, jnp.exp(-(t0 * t0)), 0.0)
    e_fac = jnp.exp((2.0 * step * inv_sigma * inv_sigma) * r)
    fb = fb_ref[0]
    w = w_ref[...]

    p_slabs = []
    f_slabs = []
    for m in range(_NB):
        if m > 0:
            rho = math.exp(-((2 * m - 1)) * (step * inv_sigma) ** 2)
            phi = phi * (e_fac * rho)
        p_slabs.append(phi)
        f_slabs.append(
            jax.lax.dot_general(
                fb, w[m], (((1,), (1,)), ((), ())),
                preferred_element_type=jnp.float32,
            )
        )
    p_cat = jnp.concatenate(p_slabs, axis=1).astype(jnp.bfloat16)
    f_cat = jnp.concatenate(f_slabs, axis=0).astype(jnp.bfloat16)
    acc = jax.lax.dot_general(
        p_cat, f_cat, (((1,), (0,)), ((), ())), preferred_element_type=jnp.float32
    )

    @pl.when(j == 0)
    def _():
        out_ref[0] = acc

    @pl.when(j != 0)
    def _():
        out_ref[0] += acc


@functools.partial(jax.jit, static_argnums=())
def kernel(features, geometry, W):
    batch, n, d_in = features.shape
    d_out = W.shape[1]
    gt = jnp.transpose(geometry, (0, 2, 1))
    gt = jnp.concatenate([gt, jnp.zeros((batch, 5, n), jnp.float32)], axis=1)

    grid = (batch, n // _BM, n // _BN)
    out = pl.pallas_call(
        _tile_body,
        grid=grid,
        in_specs=[
            pl.BlockSpec((1, 8, _BM), lambda z, i, j: (z, 0, i)),
            pl.BlockSpec((1, 8, _BN), lambda z, i, j: (z, 0, j)),
            pl.BlockSpec((1, _BN, d_in), lambda z, i, j: (z, j, 0)),
            pl.BlockSpec((_NB, d_out, d_in), lambda z, i, j: (0, 0, 0)),
        ],
        out_specs=pl.BlockSpec((1, _BM, d_out), lambda z, i, j: (z, i, 0)),
        out_shape=jax.ShapeDtypeStruct((batch, n, d_out), jnp.float32),
    )(gt, gt, features, W)
    return out


# speedup vs baseline: 1.0075x; 1.0075x over previous
import functools
import math

import jax
import jax.numpy as jnp
from jax.experimental import pallas as pl

_RADIUS = 0.1
_NB = 8
_BM = 256
_BN = 256


def _tile_body(gat_ref, gbt_ref, fb_ref, w_ref, out_ref):
    j = pl.program_id(2)

    ga = gat_ref[0]
    gb = gbt_ref[0]
    d2 = jnp.zeros((ga.shape[1], gb.shape[1]), jnp.float32)
    for c in range(3):
        dc = ga[c][:, None] - gb[c][None, :]
        d2 = d2 + dc * dc
    mask = d2 < _RADIUS * _RADIUS
    r = jnp.minimum(jnp.sqrt(d2 + 1e-12), 1.25 * _RADIUS)

    inv_sigma = _NB / _RADIUS
    step = _RADIUS / (_NB - 1)
    t0 = r * inv_sigma
    phi = jnp.where(mask, jnp.exp(-(t0 * t0)), 0.0)
    e_fac = jnp.exp((2.0 * step * inv_sigma * inv_sigma) * r)
    fb = fb_ref[0]
    w = w_ref[...]

    p_slabs = []
    f_slabs = []
    for m in range(_NB):
        if m > 0:
            rho = math.exp(-((2 * m - 1)) * (step * inv_sigma) ** 2)
            phi = phi * (e_fac * rho)
        p_slabs.append(phi)
        f_slabs.append(
            jax.lax.dot_general(
                fb, w[m], (((1,), (1,)), ((), ())),
                preferred_element_type=jnp.float32,
            )
        )
    p_cat = jnp.concatenate(p_slabs, axis=1)
    f_cat = jnp.concatenate(f_slabs, axis=0)
    acc = jax.lax.dot_general(
        p_cat, f_cat, (((1,), (0,)), ((), ())), preferred_element_type=jnp.float32
    )

    @pl.when(j == 0)
    def _():
        out_ref[0] = acc

    @pl.when(j != 0)
    def _():
        out_ref[0] += acc


@functools.partial(jax.jit, static_argnums=())
def kernel(features, geometry, W):
    batch, n, d_in = features.shape
    d_out = W.shape[1]
    gt = jnp.transpose(geometry, (0, 2, 1))
    gt = jnp.concatenate([gt, jnp.zeros((batch, 5, n), jnp.float32)], axis=1)

    grid = (batch, n // _BM, n // _BN)
    out = pl.pallas_call(
        _tile_body,
        grid=grid,
        in_specs=[
            pl.BlockSpec((1, 8, _BM), lambda z, i, j: (z, 0, i)),
            pl.BlockSpec((1, 8, _BN), lambda z, i, j: (z, 0, j)),
            pl.BlockSpec((1, _BN, d_in), lambda z, i, j: (z, j, 0)),
            pl.BlockSpec((_NB, d_out, d_in), lambda z, i, j: (0, 0, 0)),
        ],
        out_specs=pl.BlockSpec((1, _BM, d_out), lambda z, i, j: (z, i, 0)),
        out_shape=jax.ShapeDtypeStruct((batch, n, d_out), jnp.float32),
    )(gt, gt, features, W)
    return out


# speedup vs baseline: 1.2790x; 1.2695x over previous
import functools
import math

import jax
import jax.numpy as jnp
from jax.experimental import pallas as pl

_RADIUS = 0.1
_NB = 8
_BM = 512
_BN = 256


def _tile_body(gat_ref, gbt_ref, fb_ref, w_ref, out_ref):
    j = pl.program_id(2)

    ga = gat_ref[0]
    gb = gbt_ref[0]
    d2 = jnp.zeros((ga.shape[1], gb.shape[1]), jnp.float32)
    for c in range(3):
        dc = ga[c][:, None] - gb[c][None, :]
        d2 = d2 + dc * dc
    mask = d2 < _RADIUS * _RADIUS
    r = jnp.minimum(jnp.sqrt(d2 + 1e-12), 1.25 * _RADIUS)

    inv_sigma = _NB / _RADIUS
    step = _RADIUS / (_NB - 1)
    t0 = r * inv_sigma
    phi = jnp.where(mask, jnp.exp(-(t0 * t0)), 0.0)
    e_fac = jnp.exp((2.0 * step * inv_sigma * inv_sigma) * r)
    fb = fb_ref[0]
    w = w_ref[...]

    p_slabs = []
    f_slabs = []
    for m in range(_NB):
        if m > 0:
            rho = math.exp(-((2 * m - 1)) * (step * inv_sigma) ** 2)
            phi = phi * (e_fac * rho)
        p_slabs.append(phi)
        f_slabs.append(
            jax.lax.dot_general(
                fb, w[m], (((1,), (1,)), ((), ())),
                preferred_element_type=jnp.float32,
            )
        )
    p_cat = jnp.concatenate(p_slabs, axis=1)
    f_cat = jnp.concatenate(f_slabs, axis=0)
    acc = jax.lax.dot_general(
        p_cat, f_cat, (((1,), (0,)), ((), ())), preferred_element_type=jnp.float32
    )

    @pl.when(j == 0)
    def _():
        out_ref[0] = acc

    @pl.when(j != 0)
    def _():
        out_ref[0] += acc


@functools.partial(jax.jit, static_argnums=())
def kernel(features, geometry, W):
    batch, n, d_in = features.shape
    d_out = W.shape[1]
    gt = jnp.transpose(geometry, (0, 2, 1))
    gt = jnp.concatenate([gt, jnp.zeros((batch, 5, n), jnp.float32)], axis=1)

    grid = (batch, n // _BM, n // _BN)
    out = pl.pallas_call(
        _tile_body,
        grid=grid,
        in_specs=[
            pl.BlockSpec((1, 8, _BM), lambda z, i, j: (z, 0, i)),
            pl.BlockSpec((1, 8, _BN), lambda z, i, j: (z, 0, j)),
            pl.BlockSpec((1, _BN, d_in), lambda z, i, j: (z, j, 0)),
            pl.BlockSpec((_NB, d_out, d_in), lambda z, i, j: (0, 0, 0)),
        ],
        out_specs=pl.BlockSpec((1, _BM, d_out), lambda z, i, j: (z, i, 0)),
        out_shape=jax.ShapeDtypeStruct((batch, n, d_out), jnp.float32),
    )(gt, gt, features, W)
    return out


# speedup vs baseline: 1.4629x; 1.1438x over previous
import functools
import math

import jax
import jax.numpy as jnp
from jax.experimental import pallas as pl

_RADIUS = 0.1
_NB = 8
_BM = 1024
_BN = 256


def _tile_body(gat_ref, gbt_ref, fb_ref, w_ref, out_ref):
    j = pl.program_id(2)

    ga = gat_ref[0]
    gb = gbt_ref[0]
    d2 = jnp.zeros((ga.shape[1], gb.shape[1]), jnp.float32)
    for c in range(3):
        dc = ga[c][:, None] - gb[c][None, :]
        d2 = d2 + dc * dc
    mask = d2 < _RADIUS * _RADIUS
    r = jnp.minimum(jnp.sqrt(d2 + 1e-12), 1.25 * _RADIUS)

    inv_sigma = _NB / _RADIUS
    step = _RADIUS / (_NB - 1)
    t0 = r * inv_sigma
    phi = jnp.where(mask, jnp.exp(-(t0 * t0)), 0.0)
    e_fac = jnp.exp((2.0 * step * inv_sigma * inv_sigma) * r)
    fb = fb_ref[0]
    w = w_ref[...]

    p_slabs = []
    f_slabs = []
    for m in range(_NB):
        if m > 0:
            rho = math.exp(-((2 * m - 1)) * (step * inv_sigma) ** 2)
            phi = phi * (e_fac * rho)
        p_slabs.append(phi)
        f_slabs.append(
            jax.lax.dot_general(
                fb, w[m], (((1,), (1,)), ((), ())),
                preferred_element_type=jnp.float32,
            )
        )
    p_cat = jnp.concatenate(p_slabs, axis=1)
    f_cat = jnp.concatenate(f_slabs, axis=0)
    acc = jax.lax.dot_general(
        p_cat, f_cat, (((1,), (0,)), ((), ())), preferred_element_type=jnp.float32
    )

    @pl.when(j == 0)
    def _():
        out_ref[0] = acc

    @pl.when(j != 0)
    def _():
        out_ref[0] += acc


@functools.partial(jax.jit, static_argnums=())
def kernel(features, geometry, W):
    batch, n, d_in = features.shape
    d_out = W.shape[1]
    gt = jnp.transpose(geometry, (0, 2, 1))
    gt = jnp.concatenate([gt, jnp.zeros((batch, 5, n), jnp.float32)], axis=1)

    grid = (batch, n // _BM, n // _BN)
    out = pl.pallas_call(
        _tile_body,
        grid=grid,
        in_specs=[
            pl.BlockSpec((1, 8, _BM), lambda z, i, j: (z, 0, i)),
            pl.BlockSpec((1, 8, _BN), lambda z, i, j: (z, 0, j)),
            pl.BlockSpec((1, _BN, d_in), lambda z, i, j: (z, j, 0)),
            pl.BlockSpec((_NB, d_out, d_in), lambda z, i, j: (0, 0, 0)),
        ],
        out_specs=pl.BlockSpec((1, _BM, d_out), lambda z, i, j: (z, i, 0)),
        out_shape=jax.ShapeDtypeStruct((batch, n, d_out), jnp.float32),
    )(gt, gt, features, W)
    return out


# speedup vs baseline: 1.5642x; 1.0692x over previous
import functools
import math

import jax
import jax.numpy as jnp
from jax.experimental import pallas as pl

_RADIUS = 0.1
_NB = 8
_BM = 2048
_BN = 256


def _tile_body(gat_ref, gbt_ref, fb_ref, w_ref, out_ref):
    j = pl.program_id(2)

    ga = gat_ref[0]
    gb = gbt_ref[0]
    d2 = jnp.zeros((ga.shape[1], gb.shape[1]), jnp.float32)
    for c in range(3):
        dc = ga[c][:, None] - gb[c][None, :]
        d2 = d2 + dc * dc
    mask = d2 < _RADIUS * _RADIUS
    r = jnp.minimum(jnp.sqrt(d2 + 1e-12), 1.25 * _RADIUS)

    inv_sigma = _NB / _RADIUS
    step = _RADIUS / (_NB - 1)
    t0 = r * inv_sigma
    phi = jnp.where(mask, jnp.exp(-(t0 * t0)), 0.0)
    e_fac = jnp.exp((2.0 * step * inv_sigma * inv_sigma) * r)
    fb = fb_ref[0]
    w = w_ref[...]

    p_slabs = []
    f_slabs = []
    for m in range(_NB):
        if m > 0:
            rho = math.exp(-((2 * m - 1)) * (step * inv_sigma) ** 2)
            phi = phi * (e_fac * rho)
        p_slabs.append(phi)
        f_slabs.append(
            jax.lax.dot_general(
                fb, w[m], (((1,), (1,)), ((), ())),
                preferred_element_type=jnp.float32,
            )
        )
    p_cat = jnp.concatenate(p_slabs, axis=1)
    f_cat = jnp.concatenate(f_slabs, axis=0)
    acc = jax.lax.dot_general(
        p_cat, f_cat, (((1,), (0,)), ((), ())), preferred_element_type=jnp.float32
    )

    @pl.when(j == 0)
    def _():
        out_ref[0] = acc

    @pl.when(j != 0)
    def _():
        out_ref[0] += acc


@functools.partial(jax.jit, static_argnums=())
def kernel(features, geometry, W):
    batch, n, d_in = features.shape
    d_out = W.shape[1]
    gt = jnp.transpose(geometry, (0, 2, 1))
    gt = jnp.concatenate([gt, jnp.zeros((batch, 5, n), jnp.float32)], axis=1)

    grid = (batch, n // _BM, n // _BN)
    out = pl.pallas_call(
        _tile_body,
        grid=grid,
        in_specs=[
            pl.BlockSpec((1, 8, _BM), lambda z, i, j: (z, 0, i)),
            pl.BlockSpec((1, 8, _BN), lambda z, i, j: (z, 0, j)),
            pl.BlockSpec((1, _BN, d_in), lambda z, i, j: (z, j, 0)),
            pl.BlockSpec((_NB, d_out, d_in), lambda z, i, j: (0, 0, 0)),
        ],
        out_specs=pl.BlockSpec((1, _BM, d_out), lambda z, i, j: (z, i, 0)),
        out_shape=jax.ShapeDtypeStruct((batch, n, d_out), jnp.float32),
    )(gt, gt, features, W)
    return out


# speedup vs baseline: 1.6148x; 1.0323x over previous
import functools
import math

import jax
import jax.numpy as jnp
from jax.experimental import pallas as pl

_RADIUS = 0.1
_NB = 8
_BM = 2048
_BN = 512


def _tile_body(gat_ref, gbt_ref, fb_ref, w_ref, out_ref):
    j = pl.program_id(2)

    ga = gat_ref[0]
    gb = gbt_ref[0]
    d2 = jnp.zeros((ga.shape[1], gb.shape[1]), jnp.float32)
    for c in range(3):
        dc = ga[c][:, None] - gb[c][None, :]
        d2 = d2 + dc * dc
    mask = d2 < _RADIUS * _RADIUS
    r = jnp.minimum(jnp.sqrt(d2 + 1e-12), 1.25 * _RADIUS)

    inv_sigma = _NB / _RADIUS
    step = _RADIUS / (_NB - 1)
    t0 = r * inv_sigma
    phi = jnp.where(mask, jnp.exp(-(t0 * t0)), 0.0)
    e_fac = jnp.exp((2.0 * step * inv_sigma * inv_sigma) * r)
    fb = fb_ref[0]
    w = w_ref[...]

    p_slabs = []
    f_slabs = []
    for m in range(_NB):
        if m > 0:
            rho = math.exp(-((2 * m - 1)) * (step * inv_sigma) ** 2)
            phi = phi * (e_fac * rho)
        p_slabs.append(phi)
        f_slabs.append(
            jax.lax.dot_general(
                fb, w[m], (((1,), (1,)), ((), ())),
                preferred_element_type=jnp.float32,
            )
        )
    p_cat = jnp.concatenate(p_slabs, axis=1)
    f_cat = jnp.concatenate(f_slabs, axis=0)
    acc = jax.lax.dot_general(
        p_cat, f_cat, (((1,), (0,)), ((), ())), preferred_element_type=jnp.float32
    )

    @pl.when(j == 0)
    def _():
        out_ref[0] = acc

    @pl.when(j != 0)
    def _():
        out_ref[0] += acc


@functools.partial(jax.jit, static_argnums=())
def kernel(features, geometry, W):
    batch, n, d_in = features.shape
    d_out = W.shape[1]
    gt = jnp.transpose(geometry, (0, 2, 1))
    gt = jnp.concatenate([gt, jnp.zeros((batch, 5, n), jnp.float32)], axis=1)

    grid = (batch, n // _BM, n // _BN)
    out = pl.pallas_call(
        _tile_body,
        grid=grid,
        in_specs=[
            pl.BlockSpec((1, 8, _BM), lambda z, i, j: (z, 0, i)),
            pl.BlockSpec((1, 8, _BN), lambda z, i, j: (z, 0, j)),
            pl.BlockSpec((1, _BN, d_in), lambda z, i, j: (z, j, 0)),
            pl.BlockSpec((_NB, d_out, d_in), lambda z, i, j: (0, 0, 0)),
        ],
        out_specs=pl.BlockSpec((1, _BM, d_out), lambda z, i, j: (z, i, 0)),
        out_shape=jax.ShapeDtypeStruct((batch, n, d_out), jnp.float32),
    )(gt, gt, features, W)
    return out
